# baseline (device time: 62752 ns/iter reference)
import jax
import jax.numpy as jnp
from jax import lax
from jax.experimental import pallas as pl
from jax.experimental.pallas import tpu as pltpu

N_DEV = 4
SEND_ORDER = (1, 3, 2)


def kernel(x, w_mat):
    m_per, k = x.shape
    _, n = w_mat.shape
    n_per = n // N_DEV
    x_rows = 256
    x_chunks = m_per // x_rows

    def body(x_hbm, w_hbm, out_ref,
             xf_ref, xbf_ref, wf_ref, wbf_ref,
             sendq_ref, recvq_ref, sscale_ref, rscale_ref,
             xdma_sems, wdma_sems, send_sems, recv_sems):
        me = lax.axis_index("i")
        block_js = [(me + d) % N_DEV for d in SEND_ORDER] + [me]

        def x_dma(r, buf):
            return pltpu.make_async_copy(
                x_hbm.at[pl.ds(r * x_rows, x_rows), :],
                xf_ref.at[buf], xdma_sems.at[buf])

        def w_dma(s):
            return pltpu.make_async_copy(
                w_hbm.at[:, pl.ds(block_js[s] * n_per, n_per)],
                wf_ref.at[s % 2], wdma_sems.at[s % 2])

        def data_rdma(s, dev):
            return pltpu.make_async_remote_copy(
                src_ref=sendq_ref.at[s], dst_ref=recvq_ref.at[s],
                send_sem=send_sems.at[s, 0], recv_sem=recv_sems.at[s, 0],
                device_id=(dev,), device_id_type=pl.DeviceIdType.MESH)

        def scale_rdma(s, dev):
            return pltpu.make_async_remote_copy(
                src_ref=sscale_ref.at[s], dst_ref=rscale_ref.at[s],
                send_sem=send_sems.at[s, 1], recv_sem=recv_sems.at[s, 1],
                device_id=(dev,), device_id_type=pl.DeviceIdType.MESH)

        def drain_slot(s):
            p = (me - SEND_ORDER[s]) % N_DEV
            scale_rdma(s, p).wait_recv()
            data_rdma(s, p).wait_recv()
            out_ref[pl.ds(p * m_per, m_per), :] = (
                recvq_ref[s].astype(jnp.float32) * rscale_ref[s])

        x_dma(0, 0).start()
        x_dma(1, 1).start()
        w_dma(0).start()
        w_dma(1).start()

        barrier_sem = pltpu.get_barrier_semaphore()
        for d in range(N_DEV):
            @pl.when(me != d)
            def _():
                pl.semaphore_signal(
                    barrier_sem, inc=1,
                    device_id=(d,), device_id_type=pl.DeviceIdType.MESH)
        pl.semaphore_wait(barrier_sem, N_DEV - 1)

        for r in range(x_chunks):
            x_dma(r, r % 2).wait()
            if r + 2 < x_chunks:
                x_dma(r + 2, r % 2).start()
            xbf_ref[pl.ds(r * x_rows, x_rows), :] = (
                xf_ref[r % 2].astype(jnp.bfloat16))

        w_dma(0).wait()
        wbf_ref[0] = wf_ref[0].astype(jnp.bfloat16)

        for s in range(3):
            w_dma(s + 1).wait()
            if s + 2 <= 3:
                w_dma(s + 2).start()
            y = jnp.dot(xbf_ref[...], wbf_ref[s % 3],
                        preferred_element_type=jnp.float32)
            y = jnp.maximum(y, 0.0)
            rowmax = jnp.max(y, axis=1, keepdims=True)
            inv = 255.0 / jnp.maximum(rowmax, 1e-30)
            sscale_ref[s] = rowmax * (1.0 / 255.0)
            sendq_ref[s] = jnp.clip(
                jnp.round(y * inv), 0.0, 255.0).astype(jnp.uint8)
            scale_rdma(s, block_js[s]).start()
            data_rdma(s, block_js[s]).start()
            wbf_ref[(s + 1) % 3] = (
                wf_ref[(s + 1) % 2].astype(jnp.bfloat16))

        drain_slot(0)

        y = jnp.dot(xbf_ref[...], wbf_ref[0],
                    preferred_element_type=jnp.float32)
        out_ref[pl.ds(me * m_per, m_per), :] = jnp.maximum(y, 0.0)

        drain_slot(1)
        drain_slot(2)

        for s in range(3):
            data_rdma(s, block_js[s]).wait_send()
            scale_rdma(s, block_js[s]).wait_send()

    return pl.pallas_call(
        body,
        out_shape=jax.ShapeDtypeStruct((N_DEV * m_per, n_per), jnp.float32),
        in_specs=[
            pl.BlockSpec(memory_space=pl.ANY),
            pl.BlockSpec(memory_space=pl.ANY),
        ],
        out_specs=pl.BlockSpec(memory_space=pltpu.VMEM),
        scratch_shapes=[
            pltpu.VMEM((2, x_rows, k), jnp.float32),
            pltpu.VMEM((m_per, k), jnp.bfloat16),
            pltpu.VMEM((2, k, n_per), jnp.float32),
            pltpu.VMEM((3, k, n_per), jnp.bfloat16),
            pltpu.VMEM((3, m_per, n_per), jnp.uint8),
            pltpu.VMEM((3, m_per, n_per), jnp.uint8),
            pltpu.VMEM((3, m_per, 1), jnp.float32),
            pltpu.VMEM((3, m_per, 1), jnp.float32),
            pltpu.SemaphoreType.DMA((2,)),
            pltpu.SemaphoreType.DMA((2,)),
            pltpu.SemaphoreType.DMA((3, 2)),
            pltpu.SemaphoreType.DMA((3, 2)),
        ],
        compiler_params=pltpu.CompilerParams(
            collective_id=0,
            vmem_limit_bytes=63 * 1024 * 1024,
        ),
    )(x, w_mat)


# device time: 57279 ns/iter; 1.0955x vs baseline; 1.0955x over previous
import jax
import jax.numpy as jnp
from jax import lax
from jax.experimental import pallas as pl
from jax.experimental.pallas import tpu as pltpu

N_DEV = 4
SEND_ORDER = (1, 3, 2)


def kernel(x, w_mat):
    m_per, k = x.shape
    _, n = w_mat.shape
    n_per = n // N_DEV
    n_half = n_per // 2
    x_rows = 256
    x_chunks = m_per // x_rows

    chunks = [(s, h) for h in (0, 1) for s in (0, 1, 2)] + [(3, 0), (3, 1)]

    def body(x_hbm, w_hbm, out_ref,
             xf_ref, xbf_ref, wf_ref, wbf_ref, send_ref, recv_ref,
             xdma_sems, wdma_sems, send_sems, recv_sems):
        me = lax.axis_index("i")
        block_js = [(me + d) % N_DEV for d in SEND_ORDER] + [me]

        def x_dma(r, buf):
            return pltpu.make_async_copy(
                x_hbm.at[pl.ds(r * x_rows, x_rows), :],
                xf_ref.at[buf], xdma_sems.at[buf])

        def w_dma(c):
            s, h = chunks[c]
            return pltpu.make_async_copy(
                w_hbm.at[:, pl.ds(block_js[s] * n_per + h * n_half, n_half)],
                wf_ref.at[c % 2], wdma_sems.at[c % 2])

        def piece_rdma(s, h, dev):
            return pltpu.make_async_remote_copy(
                src_ref=send_ref.at[s, h], dst_ref=recv_ref.at[s, h],
                send_sem=send_sems.at[s, h], recv_sem=recv_sems.at[s, h],
                device_id=(dev,), device_id_type=pl.DeviceIdType.MESH)

        x_dma(0, 0).start()
        x_dma(1, 1).start()
        w_dma(0).start()
        w_dma(1).start()

        barrier_sem = pltpu.get_barrier_semaphore()
        for d in range(N_DEV):
            @pl.when(me != d)
            def _():
                pl.semaphore_signal(
                    barrier_sem, inc=1,
                    device_id=(d,), device_id_type=pl.DeviceIdType.MESH)
        pl.semaphore_wait(barrier_sem, N_DEV - 1)

        for r in range(x_chunks):
            x_dma(r, r % 2).wait()
            if r + 2 < x_chunks:
                x_dma(r + 2, r % 2).start()
            xbf_ref[pl.ds(r * x_rows, x_rows), :] = (
                xf_ref[r % 2].astype(jnp.bfloat16))

        for c, (s, h) in enumerate(chunks):
            w_dma(c).wait()
            if c + 2 < len(chunks):
                w_dma(c + 2).start()
            wbf_ref[c % 2] = wf_ref[c % 2].astype(jnp.bfloat16)
            y = jnp.dot(xbf_ref[...], wbf_ref[c % 2],
                        preferred_element_type=jnp.float32)
            y = jnp.maximum(y, 0.0)
            if s < 3:
                send_ref[s, h] = y.astype(jnp.bfloat16)
                piece_rdma(s, h, block_js[s]).start()
            else:
                out_ref[pl.ds(me * m_per, m_per),
                        h * n_half:(h + 1) * n_half] = y

        for s in range(3):
            p = (me - SEND_ORDER[s]) % N_DEV
            for h in (0, 1):
                piece_rdma(s, h, p).wait_recv()
                out_ref[pl.ds(p * m_per, m_per),
                        h * n_half:(h + 1) * n_half] = (
                    recv_ref[s, h].astype(jnp.float32))

        for s in range(3):
            for h in (0, 1):
                piece_rdma(s, h, block_js[s]).wait_send()

    return pl.pallas_call(
        body,
        out_shape=jax.ShapeDtypeStruct((N_DEV * m_per, n_per), jnp.float32),
        in_specs=[
            pl.BlockSpec(memory_space=pl.ANY),
            pl.BlockSpec(memory_space=pl.ANY),
        ],
        out_specs=pl.BlockSpec(memory_space=pltpu.VMEM),
        scratch_shapes=[
            pltpu.VMEM((2, x_rows, k), jnp.float32),
            pltpu.VMEM((m_per, k), jnp.bfloat16),
            pltpu.VMEM((2, k, n_half), jnp.float32),
            pltpu.VMEM((2, k, n_half), jnp.bfloat16),
            pltpu.VMEM((3, 2, m_per, n_half), jnp.bfloat16),
            pltpu.VMEM((3, 2, m_per, n_half), jnp.bfloat16),
            pltpu.SemaphoreType.DMA((2,)),
            pltpu.SemaphoreType.DMA((2,)),
            pltpu.SemaphoreType.DMA((3, 2)),
            pltpu.SemaphoreType.DMA((3, 2)),
        ],
        compiler_params=pltpu.CompilerParams(
            collective_id=0,
            vmem_limit_bytes=60 * 1024 * 1024,
        ),
    )(x, w_mat)


# device time: 57149 ns/iter; 1.0980x vs baseline; 1.0023x over previous
import jax
import jax.numpy as jnp
from jax import lax
from jax.experimental import pallas as pl
from jax.experimental.pallas import tpu as pltpu

N_DEV = 4
SEND_ORDER = (1, 3, 2)


def kernel(x, w_mat):
    m_per, k = x.shape
    _, n = w_mat.shape
    n_per = n // N_DEV
    n_half = n_per // 2
    x_rows = 256
    x_chunks = m_per // x_rows

    chunks = [(s, h) for h in (0, 1) for s in (0, 1, 2)] + [(3, 0), (3, 1)]

    def body(x_hbm, w_hbm, out_ref,
             xf_ref, xbf_ref, wf_ref, wbf_ref, send_ref, recv_ref,
             xdma_sems, wdma_sems, send_sems, recv_sems):
        me = lax.axis_index("i")
        block_js = [(me + d) % N_DEV for d in SEND_ORDER] + [me]

        def x_dma(r, buf):
            return pltpu.make_async_copy(
                x_hbm.at[pl.ds(r * x_rows, x_rows), :],
                xf_ref.at[buf], xdma_sems.at[buf])

        def w_dma(c):
            s, h = chunks[c]
            return pltpu.make_async_copy(
                w_hbm.at[:, pl.ds(block_js[s] * n_per + h * n_half, n_half)],
                wf_ref.at[c % 2], wdma_sems.at[c % 2])

        def piece_rdma(s, h, dev):
            return pltpu.make_async_remote_copy(
                src_ref=send_ref.at[s, h], dst_ref=recv_ref.at[s, h],
                send_sem=send_sems.at[s, h], recv_sem=recv_sems.at[s, h],
                device_id=(dev,), device_id_type=pl.DeviceIdType.MESH)

        x_dma(0, 0).start()
        x_dma(1, 1).start()
        w_dma(0).start()
        w_dma(1).start()

        barrier_sem = pltpu.get_barrier_semaphore()
        for d in range(N_DEV):
            @pl.when(me != d)
            def _():
                pl.semaphore_signal(
                    barrier_sem, inc=1,
                    device_id=(d,), device_id_type=pl.DeviceIdType.MESH)
        pl.semaphore_wait(barrier_sem, N_DEV - 1)

        for r in range(x_chunks):
            x_dma(r, r % 2).wait()
            if r + 2 < x_chunks:
                x_dma(r + 2, r % 2).start()
            xbf_ref[pl.ds(r * x_rows, x_rows), :] = (
                xf_ref[r % 2].astype(jnp.bfloat16))

        for c, (s, h) in enumerate(chunks):
            w_dma(c).wait()
            wbf_ref[c % 2] = wf_ref[c % 2].astype(jnp.bfloat16)
            if c + 2 < len(chunks):
                w_dma(c + 2).start()
            y = jnp.dot(xbf_ref[...], wbf_ref[c % 2],
                        preferred_element_type=jnp.float32)
            y = jnp.maximum(y, 0.0)
            if s < 3:
                send_ref[s, h] = y.astype(jnp.bfloat16)
                piece_rdma(s, h, block_js[s]).start()
            else:
                out_ref[pl.ds(me * m_per, m_per),
                        h * n_half:(h + 1) * n_half] = y

        for s in range(3):
            p = (me - SEND_ORDER[s]) % N_DEV
            for h in (0, 1):
                piece_rdma(s, h, p).wait_recv()
                out_ref[pl.ds(p * m_per, m_per),
                        h * n_half:(h + 1) * n_half] = (
                    recv_ref[s, h].astype(jnp.float32))

        for s in range(3):
            for h in (0, 1):
                piece_rdma(s, h, block_js[s]).wait_send()

    return pl.pallas_call(
        body,
        out_shape=jax.ShapeDtypeStruct((N_DEV * m_per, n_per), jnp.float32),
        in_specs=[
            pl.BlockSpec(memory_space=pl.ANY),
            pl.BlockSpec(memory_space=pl.ANY),
        ],
        out_specs=pl.BlockSpec(memory_space=pltpu.VMEM),
        scratch_shapes=[
            pltpu.VMEM((2, x_rows, k), jnp.float32),
            pltpu.VMEM((m_per, k), jnp.bfloat16),
            pltpu.VMEM((2, k, n_half), jnp.float32),
            pltpu.VMEM((2, k, n_half), jnp.bfloat16),
            pltpu.VMEM((3, 2, m_per, n_half), jnp.bfloat16),
            pltpu.VMEM((3, 2, m_per, n_half), jnp.bfloat16),
            pltpu.SemaphoreType.DMA((2,)),
            pltpu.SemaphoreType.DMA((2,)),
            pltpu.SemaphoreType.DMA((3, 2)),
            pltpu.SemaphoreType.DMA((3, 2)),
        ],
        compiler_params=pltpu.CompilerParams(
            collective_id=0,
            vmem_limit_bytes=60 * 1024 * 1024,
        ),
    )(x, w_mat)


# device time: 53737 ns/iter; 1.1678x vs baseline; 1.0635x over previous
import jax
import jax.numpy as jnp
from jax import lax
from jax.experimental import pallas as pl
from jax.experimental.pallas import tpu as pltpu

N_DEV = 4
SEND_ORDER = (1, 3, 2)


def kernel(x, w_mat):
    m_per, k = x.shape
    _, n = w_mat.shape
    n_per = n // N_DEV
    n_half = n_per // 2
    x_rows = 256
    x_chunks = m_per // x_rows

    chunks = [(s, h) for h in (0, 1) for s in (0, 1, 2)] + [(3, 0), (3, 1)]

    def body(x_hbm, w_hbm, out_ref,
             xf_ref, xbf_ref, wf_ref, wbf_ref, send_ref,
             xdma_sems, wdma_sems, send_sems, recv_sems):
        me = lax.axis_index("i")
        block_js = [(me + d) % N_DEV for d in SEND_ORDER] + [me]

        def x_dma(r, buf):
            return pltpu.make_async_copy(
                x_hbm.at[pl.ds(r * x_rows, x_rows), :],
                xf_ref.at[buf], xdma_sems.at[buf])

        def w_dma(c):
            s, h = chunks[c]
            return pltpu.make_async_copy(
                w_hbm.at[:, pl.ds(block_js[s] * n_per + h * n_half, n_half)],
                wf_ref.at[c % 2], wdma_sems.at[c % 2])

        def piece_rdma(s, h, dev, dst_rows):
            return pltpu.make_async_remote_copy(
                src_ref=send_ref.at[s, h],
                dst_ref=out_ref.at[pl.ds(dst_rows * m_per, m_per),
                                   pl.ds(h * n_half, n_half)],
                send_sem=send_sems.at[s, h], recv_sem=recv_sems.at[s, h],
                device_id=(dev,), device_id_type=pl.DeviceIdType.MESH)

        x_dma(0, 0).start()
        x_dma(1, 1).start()
        w_dma(0).start()
        w_dma(1).start()

        barrier_sem = pltpu.get_barrier_semaphore()
        for d in range(N_DEV):
            @pl.when(me != d)
            def _():
                pl.semaphore_signal(
                    barrier_sem, inc=1,
                    device_id=(d,), device_id_type=pl.DeviceIdType.MESH)
        pl.semaphore_wait(barrier_sem, N_DEV - 1)

        for r in range(x_chunks):
            x_dma(r, r % 2).wait()
            if r + 2 < x_chunks:
                x_dma(r + 2, r % 2).start()
            xbf_ref[pl.ds(r * x_rows, x_rows), :] = (
                xf_ref[r % 2].astype(jnp.bfloat16))

        for c, (s, h) in enumerate(chunks):
            w_dma(c).wait()
            wbf_ref[c % 2] = wf_ref[c % 2].astype(jnp.bfloat16)
            if c + 2 < len(chunks):
                w_dma(c + 2).start()
            y = jnp.dot(xbf_ref[...], wbf_ref[c % 2],
                        preferred_element_type=jnp.float32)
            y = jnp.maximum(y, 0.0)
            if s < 3:
                send_ref[s, h] = y.astype(jnp.bfloat16)
                piece_rdma(s, h, block_js[s], me).start()
            else:
                out_ref[pl.ds(me * m_per, m_per),
                        h * n_half:(h + 1) * n_half] = y.astype(jnp.bfloat16)

        for s in range(3):
            p = (me - SEND_ORDER[s]) % N_DEV
            for h in (0, 1):
                piece_rdma(s, h, p, p).wait_recv()

        for s in range(3):
            for h in (0, 1):
                piece_rdma(s, h, block_js[s], me).wait_send()

    return pl.pallas_call(
        body,
        out_shape=jax.ShapeDtypeStruct((N_DEV * m_per, n_per), jnp.bfloat16),
        in_specs=[
            pl.BlockSpec(memory_space=pl.ANY),
            pl.BlockSpec(memory_space=pl.ANY),
        ],
        out_specs=pl.BlockSpec(memory_space=pltpu.VMEM),
        scratch_shapes=[
            pltpu.VMEM((2, x_rows, k), jnp.float32),
            pltpu.VMEM((m_per, k), jnp.bfloat16),
            pltpu.VMEM((2, k, n_half), jnp.float32),
            pltpu.VMEM((2, k, n_half), jnp.bfloat16),
            pltpu.VMEM((3, 2, m_per, n_half), jnp.bfloat16),
            pltpu.SemaphoreType.DMA((2,)),
            pltpu.SemaphoreType.DMA((2,)),
            pltpu.SemaphoreType.DMA((3, 2)),
            pltpu.SemaphoreType.DMA((3, 2)),
        ],
        compiler_params=pltpu.CompilerParams(
            collective_id=0,
            vmem_limit_bytes=60 * 1024 * 1024,
        ),
    )(x, w_mat)
